# Initial kernel scaffold; baseline (speedup 1.0000x reference)
import jax
import jax.numpy as jnp
from jax.experimental import pallas as pl


def kernel(node_feat, edge_feat, smooth_weight, source_index, target_index, target_bincount, directed2undirected, Wg, Wv, Wo, ln_scale, ln_bias, W1_node, W2_node, W1_edge, W2_edge, W_env, node_res_weight, edge_res_weight):
    raise NotImplementedError("write your pallas kernel here")


# speedup vs baseline: 1.2472x; 1.2472x over previous
import functools

import jax
import jax.numpy as jnp
from jax.experimental import pallas as pl
from jax.experimental.pallas import tpu as pltpu

N = 10000
E_DIR = 320000
E_UND = 160000
D = 128
H = 128
NB = 7

BE = 2000
N_EBLK = E_DIR // BE


def _silu(x):
    return x * jax.nn.sigmoid(x)


def _dense_body(e0_ref, tgt_ref, src_ref, swg_ref, wgv_ref, wo_ref,
                lns_ref, lnb_ref, wenv_ref, w1e_ref, w2e_ref,
                smooth_ref, de_ref):
    e0 = e0_ref[...]
    tgt = tgt_ref[...]
    src = src_ref[...]
    wgv = wgv_ref[...]
    gv = (jnp.dot(e0, wgv[0:D], preferred_element_type=jnp.float32)
          + jnp.dot(tgt, wgv[D:2 * D], preferred_element_type=jnp.float32)
          + jnp.dot(src, wgv[2 * D:3 * D], preferred_element_type=jnp.float32))
    g = gv[:, 0:H]
    v = gv[:, H:2 * H]
    h = _silu(g) * v
    ho = jnp.dot(h, wo_ref[...], preferred_element_type=jnp.float32)
    m = jnp.mean(ho, axis=-1, keepdims=True)
    c = ho - m
    var = jnp.mean(c * c, axis=-1, keepdims=True)
    ln = c * jax.lax.rsqrt(var + 1e-5) * lns_ref[...] + lnb_ref[...]
    sw = jnp.dot(swg_ref[...], wenv_ref[...], preferred_element_type=jnp.float32)
    smooth_ref[...] = ln * sw
    de = jnp.dot(_silu(jnp.dot(ln, w1e_ref[...], preferred_element_type=jnp.float32)),
                 w2e_ref[...], preferred_element_type=jnp.float32)
    de_ref[...] = de


def _dense_call(e0, tgt, src, swg, wgv, wo, ln_scale, ln_bias, wenv16, w1e, w2e):
    eb = lambda i: (i, 0)
    wb = lambda i: (0, 0)
    return pl.pallas_call(
        _dense_body,
        grid=(N_EBLK,),
        in_specs=[
            pl.BlockSpec((BE, D), eb),
            pl.BlockSpec((BE, D), eb),
            pl.BlockSpec((BE, D), eb),
            pl.BlockSpec((BE, 16), eb),
            pl.BlockSpec((3 * D, 2 * H), wb),
            pl.BlockSpec((H, D), wb),
            pl.BlockSpec((1, D), wb),
            pl.BlockSpec((1, D), wb),
            pl.BlockSpec((16, D), wb),
            pl.BlockSpec((D, D), wb),
            pl.BlockSpec((D, D), wb),
        ],
        out_specs=[pl.BlockSpec((BE, D), eb), pl.BlockSpec((BE, D), eb)],
        out_shape=[
            jax.ShapeDtypeStruct((E_DIR, D), jnp.float32),
            jax.ShapeDtypeStruct((E_DIR, D), jnp.float32),
        ],
    )(e0, tgt, src, swg, wgv, wo, ln_scale, ln_bias, wenv16, w1e, w2e)


def _node_body(refine_ref, nf_ref, w1_ref, w2_ref, nrw_ref, out_ref):
    r = refine_ref[...]
    d = jnp.dot(_silu(jnp.dot(r, w1_ref[...], preferred_element_type=jnp.float32)),
                w2_ref[...], preferred_element_type=jnp.float32)
    out_ref[...] = d + nrw_ref[...] * nf_ref[...]


def _node_call(refine, node_feat, w1, w2, nrw):
    nb = lambda i: (i, 0)
    wb = lambda i: (0, 0)
    return pl.pallas_call(
        _node_body,
        grid=(5,),
        in_specs=[
            pl.BlockSpec((N // 5, D), nb),
            pl.BlockSpec((N // 5, D), nb),
            pl.BlockSpec((D, D), wb),
            pl.BlockSpec((D, D), wb),
            pl.BlockSpec((1, D), wb),
        ],
        out_specs=pl.BlockSpec((N // 5, D), nb),
        out_shape=jax.ShapeDtypeStruct((N, D), jnp.float32),
    )(refine, node_feat, w1, w2, nrw)


def _edge_fin_body(ssum_ref, cnt_ref, ef_ref, erw_ref, out_ref):
    c = jnp.maximum(cnt_ref[...], 1.0)
    out_ref[...] = ssum_ref[...] / c + erw_ref[...] * ef_ref[...]


def _edge_fin_call(ssum, counts, edge_feat, erw):
    BU = 8000
    ub = lambda i: (i, 0)
    wb = lambda i: (0, 0)
    return pl.pallas_call(
        _edge_fin_body,
        grid=(E_UND // BU,),
        in_specs=[
            pl.BlockSpec((BU, D), ub),
            pl.BlockSpec((BU, 1), ub),
            pl.BlockSpec((BU, D), ub),
            pl.BlockSpec((1, D), wb),
        ],
        out_specs=pl.BlockSpec((BU, D), ub),
        out_shape=jax.ShapeDtypeStruct((E_UND, D), jnp.float32),
    )(ssum, counts, edge_feat, erw)


def kernel(node_feat, edge_feat, smooth_weight, source_index, target_index,
           target_bincount, directed2undirected, Wg, Wv, Wo, ln_scale, ln_bias,
           W1_node, W2_node, W1_edge, W2_edge, W_env, node_res_weight, edge_res_weight):
    wgv = jnp.concatenate([Wg, Wv], axis=1)
    wenv16 = jnp.pad(W_env, ((0, 16 - NB), (0, 0)))
    sw16 = jnp.pad(smooth_weight, ((0, 0), (0, 16 - NB)))
    lns = ln_scale.reshape(1, D)
    lnb = ln_bias.reshape(1, D)

    e0 = jnp.take(edge_feat, directed2undirected, axis=0)
    tgt = jnp.take(node_feat, target_index, axis=0)
    src = jnp.take(node_feat, source_index, axis=0)
    swg = jnp.take(sw16, directed2undirected, axis=0)

    smooth, delta_edge = _dense_call(e0, tgt, src, swg, wgv, Wo, lns, lnb,
                                     wenv16, W1_edge, W2_edge)

    refine_node = jax.ops.segment_sum(smooth, target_index, num_segments=N)
    seg_sum = jax.ops.segment_sum(delta_edge, directed2undirected, num_segments=E_UND)
    counts = jax.ops.segment_sum(jnp.ones((E_DIR, 1), jnp.float32),
                                 directed2undirected, num_segments=E_UND)

    update_node = _node_call(refine_node, node_feat, W1_node, W2_node, node_res_weight)
    update_edge = _edge_fin_call(seg_sum, counts, edge_feat, edge_res_weight)
    return (update_node, update_edge)


# speedup vs baseline: 2.0973x; 1.6816x over previous
import functools

import jax
import jax.numpy as jnp
from jax import lax
from jax.experimental import pallas as pl
from jax.experimental.pallas import tpu as pltpu
from jax.experimental.pallas import tpu_sc as plsc

N = 10000
E_DIR = 320000
E_UND = 160000
D = 128
H = 128
NB = 7

NC = 2
NS = 16
NW = NC * NS

BE = 2000
N_EBLK = E_DIR // BE


def _silu(x):
    return x * jax.nn.sigmoid(x)


EPW = E_DIR // NW
CI = 1000
CR = 200
N_PAD = 10240
NPT = N_PAD // NS

_g_mesh = plsc.VectorSubcoreMesh(core_axis_name="c", subcore_axis_name="s")


@functools.partial(
    pl.kernel,
    out_type=[
        jax.ShapeDtypeStruct((E_DIR, D), jnp.float32),
        jax.ShapeDtypeStruct((E_DIR, D), jnp.float32),
        jax.ShapeDtypeStruct((E_DIR, D), jnp.float32),
        jax.ShapeDtypeStruct((E_DIR, D), jnp.float32),
    ],
    mesh=_g_mesh,
    scratch_types=[
        pltpu.VMEM((CI,), jnp.int32),
        pltpu.VMEM((CI,), jnp.int32),
        pltpu.VMEM((CI,), jnp.int32),
        pltpu.VMEM((CI,), jnp.int32),
        pltpu.VMEM((CR, D), jnp.float32),
        pltpu.VMEM((CR, D), jnp.float32),
        pltpu.SemaphoreType.DMA,
    ],
)
def _sc_gather(nf_hbm, ef_hbm, sw_hbm, ti_hbm, si_hbm, du_hbm, du8_hbm,
               e0_out, tgt_out, src_out, swg_out,
               ti_v, si_v, du_v, du8_v, rows_v, sw_v, sem):
    cid = lax.axis_index("c")
    sid = lax.axis_index("s")
    wid = sid * NC + cid
    base = wid * EPW

    def outer(j, _):
        off = base + j * CI
        pltpu.sync_copy(ti_hbm.at[pl.ds(off, CI)], ti_v)
        pltpu.sync_copy(si_hbm.at[pl.ds(off, CI)], si_v)
        pltpu.sync_copy(du_hbm.at[pl.ds(off, CI)], du_v)
        pltpu.sync_copy(du8_hbm.at[pl.ds(off, CI)], du8_v)

        def inner(k, _):
            o2 = off + k * CR
            pltpu.async_copy(ef_hbm.at[du_v.at[pl.ds(k * CR, CR)]], rows_v, sem).wait()
            pltpu.sync_copy(rows_v, e0_out.at[pl.ds(o2, CR)])
            pltpu.async_copy(nf_hbm.at[ti_v.at[pl.ds(k * CR, CR)]], rows_v, sem).wait()
            pltpu.sync_copy(rows_v, tgt_out.at[pl.ds(o2, CR)])
            pltpu.async_copy(nf_hbm.at[si_v.at[pl.ds(k * CR, CR)]], rows_v, sem).wait()
            pltpu.sync_copy(rows_v, src_out.at[pl.ds(o2, CR)])
            pltpu.async_copy(sw_hbm.at[du8_v.at[pl.ds(k * CR, CR)]], sw_v, sem).wait()
            pltpu.sync_copy(sw_v, swg_out.at[pl.ds(o2, CR)])
            return 0

        lax.fori_loop(0, CI // CR, inner, 0)
        return 0

    lax.fori_loop(0, EPW // CI, outer, 0)


def _dense_body(e0_ref, tgt_ref, src_ref, swg_ref, dul_ref, wgv_ref, wo_ref,
                lns_ref, lnb_ref, wenv_ref, w1e_ref, w2e_ref,
                smooth_ref, de_ref):
    e0 = e0_ref[...]
    tgt = tgt_ref[...]
    src = src_ref[...]
    wgv = wgv_ref[...]
    gv = (jnp.dot(e0, wgv[0:D], preferred_element_type=jnp.float32)
          + jnp.dot(tgt, wgv[D:2 * D], preferred_element_type=jnp.float32)
          + jnp.dot(src, wgv[2 * D:3 * D], preferred_element_type=jnp.float32))
    g = gv[:, 0:H]
    v = gv[:, H:2 * H]
    h = _silu(g) * v
    ho = jnp.dot(h, wo_ref[...], preferred_element_type=jnp.float32)
    m = jnp.mean(ho, axis=-1, keepdims=True)
    c = ho - m
    var = jnp.mean(c * c, axis=-1, keepdims=True)
    ln = c * jax.lax.rsqrt(var + 1e-5) * lns_ref[...] + lnb_ref[...]
    dul = dul_ref[...].reshape(BE, 1)
    grp = lax.broadcasted_iota(jnp.int32, (BE, D), 1) // 16
    sw_sel = jnp.where(grp == dul, swg_ref[...], 0.0)
    sw = jnp.dot(sw_sel, wenv_ref[...], preferred_element_type=jnp.float32)
    smooth_ref[...] = ln * sw
    de = jnp.dot(_silu(jnp.dot(ln, w1e_ref[...], preferred_element_type=jnp.float32)),
                 w2e_ref[...], preferred_element_type=jnp.float32)
    de_ref[...] = de


def _dense_call(e0, tgt, src, swg, dul, wgv, wo, ln_scale, ln_bias, wenvx, w1e, w2e):
    eb = lambda i: (i, 0)
    ib = lambda i: (i, 0, 0)
    wb = lambda i: (0, 0)
    dul3 = dul.reshape(N_EBLK, 1, BE)
    return pl.pallas_call(
        _dense_body,
        grid=(N_EBLK,),
        in_specs=[
            pl.BlockSpec((BE, D), eb),
            pl.BlockSpec((BE, D), eb),
            pl.BlockSpec((BE, D), eb),
            pl.BlockSpec((BE, D), eb),
            pl.BlockSpec((1, 1, BE), ib),
            pl.BlockSpec((3 * D, 2 * H), wb),
            pl.BlockSpec((H, D), wb),
            pl.BlockSpec((1, D), wb),
            pl.BlockSpec((1, D), wb),
            pl.BlockSpec((D, D), wb),
            pl.BlockSpec((D, D), wb),
            pl.BlockSpec((D, D), wb),
        ],
        out_specs=[pl.BlockSpec((BE, D), eb), pl.BlockSpec((BE, D), eb)],
        out_shape=[
            jax.ShapeDtypeStruct((E_DIR, D), jnp.float32),
            jax.ShapeDtypeStruct((E_DIR, D), jnp.float32),
        ],
    )(e0, tgt, src, swg, dul3, wgv, wo, ln_scale, ln_bias, wenvx, w1e, w2e)


def _node_body(refine_ref, nf_ref, w1_ref, w2_ref, nrw_ref, out_ref):
    r = refine_ref[...]
    d = jnp.dot(_silu(jnp.dot(r, w1_ref[...], preferred_element_type=jnp.float32)),
                w2_ref[...], preferred_element_type=jnp.float32)
    out_ref[...] = d + nrw_ref[...] * nf_ref[...]


def _node_call(refine, node_feat, w1, w2, nrw):
    nb = lambda i: (i, 0)
    wb = lambda i: (0, 0)
    return pl.pallas_call(
        _node_body,
        grid=(5,),
        in_specs=[
            pl.BlockSpec((N // 5, D), nb),
            pl.BlockSpec((N // 5, D), nb),
            pl.BlockSpec((D, D), wb),
            pl.BlockSpec((D, D), wb),
            pl.BlockSpec((1, D), wb),
        ],
        out_specs=pl.BlockSpec((N // 5, D), nb),
        out_shape=jax.ShapeDtypeStruct((N, D), jnp.float32),
    )(refine, node_feat, w1, w2, nrw)


def _edge_fin_body(ssum_ref, cnt_ref, ef_ref, erw_ref, out_ref):
    c = jnp.maximum(cnt_ref[...], 1.0)
    out_ref[...] = ssum_ref[...] / c + erw_ref[...] * ef_ref[...]


def _edge_fin_call(ssum, counts, edge_feat, erw):
    BU = 8000
    ub = lambda i: (i, 0)
    wb = lambda i: (0, 0)
    return pl.pallas_call(
        _edge_fin_body,
        grid=(E_UND // BU,),
        in_specs=[
            pl.BlockSpec((BU, D), ub),
            pl.BlockSpec((BU, 1), ub),
            pl.BlockSpec((BU, D), ub),
            pl.BlockSpec((1, D), wb),
        ],
        out_specs=pl.BlockSpec((BU, D), ub),
        out_shape=jax.ShapeDtypeStruct((E_UND, D), jnp.float32),
    )(ssum, counts, edge_feat, erw)


def kernel(node_feat, edge_feat, smooth_weight, source_index, target_index,
           target_bincount, directed2undirected, Wg, Wv, Wo, ln_scale, ln_bias,
           W1_node, W2_node, W1_edge, W2_edge, W_env, node_res_weight, edge_res_weight):
    wgv = jnp.concatenate([Wg, Wv], axis=1)
    wenv16 = jnp.pad(W_env, ((0, 16 - NB), (0, 0)))
    wenvx = jnp.tile(wenv16, (8, 1))
    sw16 = jnp.pad(smooth_weight, ((0, 0), (0, 16 - NB)))
    swp8 = sw16.reshape(E_UND // 8, D)
    lns = ln_scale.reshape(1, D)
    lnb = ln_bias.reshape(1, D)
    ti1 = target_index.astype(jnp.int32)
    si1 = source_index.astype(jnp.int32)
    du1 = directed2undirected.astype(jnp.int32)
    du8 = du1 // 8
    dul = du1 % 8

    e0, tgt, src, swg = _sc_gather(node_feat, edge_feat, swp8, ti1, si1, du1, du8)

    smooth, delta_edge = _dense_call(e0, tgt, src, swg, dul, wgv, Wo, lns, lnb,
                                     wenvx, W1_edge, W2_edge)

    refine_node = jax.ops.segment_sum(smooth, target_index, num_segments=N)
    seg_sum = jax.ops.segment_sum(delta_edge, directed2undirected, num_segments=E_UND)
    counts = jax.ops.segment_sum(jnp.ones((E_DIR, 1), jnp.float32),
                                 directed2undirected, num_segments=E_UND)

    update_node = _node_call(refine_node, node_feat, W1_node, W2_node, node_res_weight)
    update_edge = _edge_fin_call(seg_sum, counts, edge_feat, edge_res_weight)
    return (update_node, update_edge)


# speedup vs baseline: 2.4115x; 1.1498x over previous
import functools

import jax
import jax.numpy as jnp
from jax import lax
from jax.experimental import pallas as pl
from jax.experimental.pallas import tpu as pltpu
from jax.experimental.pallas import tpu_sc as plsc

N = 10000
E_DIR = 320000
E_UND = 160000
D = 128
H = 128
NB = 7

NC = 2
NS = 16
NW = NC * NS

BE = 2000
N_EBLK = E_DIR // BE


def _silu(x):
    return x * jax.nn.sigmoid(x)


EPW = E_DIR // NW
CI = 1000
CR = 200
N_PAD = 10240
NPT = N_PAD // NS

_g_mesh = plsc.VectorSubcoreMesh(core_axis_name="c", subcore_axis_name="s")


@functools.partial(
    pl.kernel,
    out_type=[
        jax.ShapeDtypeStruct((E_DIR, D), jnp.float32),
        jax.ShapeDtypeStruct((E_DIR, D), jnp.float32),
        jax.ShapeDtypeStruct((E_DIR, D), jnp.float32),
        jax.ShapeDtypeStruct((E_DIR, D), jnp.float32),
    ],
    mesh=_g_mesh,
    scratch_types=[
        pltpu.VMEM((CI,), jnp.int32),
        pltpu.VMEM((CI,), jnp.int32),
        pltpu.VMEM((CI,), jnp.int32),
        pltpu.VMEM((CI,), jnp.int32),
        pltpu.VMEM((CR, D), jnp.float32),
        pltpu.VMEM((CR, D), jnp.float32),
        pltpu.SemaphoreType.DMA,
    ],
)
def _sc_gather(nf_hbm, ef_hbm, sw_hbm, ti_hbm, si_hbm, du_hbm, du8_hbm,
               e0_out, tgt_out, src_out, swg_out,
               ti_v, si_v, du_v, du8_v, rows_v, sw_v, sem):
    cid = lax.axis_index("c")
    sid = lax.axis_index("s")
    wid = sid * NC + cid
    base = wid * EPW

    def outer(j, _):
        off = base + j * CI
        pltpu.sync_copy(ti_hbm.at[pl.ds(off, CI)], ti_v)
        pltpu.sync_copy(si_hbm.at[pl.ds(off, CI)], si_v)
        pltpu.sync_copy(du_hbm.at[pl.ds(off, CI)], du_v)
        pltpu.sync_copy(du8_hbm.at[pl.ds(off, CI)], du8_v)

        def inner(k, _):
            o2 = off + k * CR
            pltpu.async_copy(ef_hbm.at[du_v.at[pl.ds(k * CR, CR)]], rows_v, sem).wait()
            pltpu.sync_copy(rows_v, e0_out.at[pl.ds(o2, CR)])
            pltpu.async_copy(nf_hbm.at[ti_v.at[pl.ds(k * CR, CR)]], rows_v, sem).wait()
            pltpu.sync_copy(rows_v, tgt_out.at[pl.ds(o2, CR)])
            pltpu.async_copy(nf_hbm.at[si_v.at[pl.ds(k * CR, CR)]], rows_v, sem).wait()
            pltpu.sync_copy(rows_v, src_out.at[pl.ds(o2, CR)])
            pltpu.async_copy(sw_hbm.at[du8_v.at[pl.ds(k * CR, CR)]], sw_v, sem).wait()
            pltpu.sync_copy(sw_v, swg_out.at[pl.ds(o2, CR)])
            return 0

        lax.fori_loop(0, CI // CR, inner, 0)
        return 0

    lax.fori_loop(0, EPW // CI, outer, 0)


EPC = E_DIR // NC
EPT = EPC // NS
NST = N_PAD // NS


@functools.partial(
    pl.kernel,
    out_type=jax.ShapeDtypeStruct((NC, N_PAD, D), jnp.float32),
    mesh=_g_mesh,
    scratch_types=[
        pltpu.VMEM((CR,), jnp.int32),
        pltpu.VMEM((CR, D), jnp.float32),
        pltpu.VMEM_SHARED((N_PAD, D), jnp.float32),
        pltpu.SemaphoreType.DMA,
    ],
)
def _sc_node_scatter(smooth_hbm, ti_hbm, zeros_hbm, out_hbm,
                     ti_v, rows_v, acc, sem):
    cid = lax.axis_index("c")
    sid = lax.axis_index("s")
    pltpu.sync_copy(zeros_hbm.at[pl.ds(sid * NST, NST)],
                    acc.at[pl.ds(sid * NST, NST)])
    plsc.subcore_barrier()
    base = cid * EPC + sid * EPT

    def step(j, _):
        off = base + j * CR
        pltpu.sync_copy(ti_hbm.at[pl.ds(off, CR)], ti_v)
        pltpu.sync_copy(smooth_hbm.at[pl.ds(off, CR)], rows_v)
        pltpu.sync_copy(rows_v, acc.at[ti_v], add=True)
        return 0

    lax.fori_loop(0, EPT // CR, step, 0)
    plsc.subcore_barrier()
    pltpu.sync_copy(acc.at[pl.ds(sid * NST, NST)],
                    out_hbm.at[cid].at[pl.ds(sid * NST, NST)])


def _dense_body(e0_ref, tgt_ref, src_ref, swg_ref, dul_ref, wgv_ref, wo_ref,
                lns_ref, lnb_ref, wenv_ref, w1e_ref, w2e_ref,
                smooth_ref, de_ref):
    e0 = e0_ref[...]
    tgt = tgt_ref[...]
    src = src_ref[...]
    wgv = wgv_ref[...]
    gv = (jnp.dot(e0, wgv[0:D], preferred_element_type=jnp.float32)
          + jnp.dot(tgt, wgv[D:2 * D], preferred_element_type=jnp.float32)
          + jnp.dot(src, wgv[2 * D:3 * D], preferred_element_type=jnp.float32))
    g = gv[:, 0:H]
    v = gv[:, H:2 * H]
    h = _silu(g) * v
    ho = jnp.dot(h, wo_ref[...], preferred_element_type=jnp.float32)
    m = jnp.mean(ho, axis=-1, keepdims=True)
    c = ho - m
    var = jnp.mean(c * c, axis=-1, keepdims=True)
    ln = c * jax.lax.rsqrt(var + 1e-5) * lns_ref[...] + lnb_ref[...]
    dul = dul_ref[...].reshape(BE, 1)
    grp = lax.broadcasted_iota(jnp.int32, (BE, D), 1) // 16
    sw_sel = jnp.where(grp == dul, swg_ref[...], 0.0)
    sw = jnp.dot(sw_sel, wenv_ref[...], preferred_element_type=jnp.float32)
    smooth_ref[...] = ln * sw
    de = jnp.dot(_silu(jnp.dot(ln, w1e_ref[...], preferred_element_type=jnp.float32)),
                 w2e_ref[...], preferred_element_type=jnp.float32)
    de_ref[...] = de


def _dense_call(e0, tgt, src, swg, dul, wgv, wo, ln_scale, ln_bias, wenvx, w1e, w2e):
    eb = lambda i: (i, 0)
    ib = lambda i: (i, 0, 0)
    wb = lambda i: (0, 0)
    dul3 = dul.reshape(N_EBLK, 1, BE)
    return pl.pallas_call(
        _dense_body,
        grid=(N_EBLK,),
        in_specs=[
            pl.BlockSpec((BE, D), eb),
            pl.BlockSpec((BE, D), eb),
            pl.BlockSpec((BE, D), eb),
            pl.BlockSpec((BE, D), eb),
            pl.BlockSpec((1, 1, BE), ib),
            pl.BlockSpec((3 * D, 2 * H), wb),
            pl.BlockSpec((H, D), wb),
            pl.BlockSpec((1, D), wb),
            pl.BlockSpec((1, D), wb),
            pl.BlockSpec((D, D), wb),
            pl.BlockSpec((D, D), wb),
            pl.BlockSpec((D, D), wb),
        ],
        out_specs=[pl.BlockSpec((BE, D), eb), pl.BlockSpec((BE, D), eb)],
        out_shape=[
            jax.ShapeDtypeStruct((E_DIR, D), jnp.float32),
            jax.ShapeDtypeStruct((E_DIR, D), jnp.float32),
        ],
    )(e0, tgt, src, swg, dul3, wgv, wo, ln_scale, ln_bias, wenvx, w1e, w2e)


def _node_body(r0_ref, r1_ref, nf_ref, w1_ref, w2_ref, nrw_ref, out_ref):
    r = r0_ref[0] + r1_ref[0]
    d = jnp.dot(_silu(jnp.dot(r, w1_ref[...], preferred_element_type=jnp.float32)),
                w2_ref[...], preferred_element_type=jnp.float32)
    out_ref[...] = d + nrw_ref[...] * nf_ref[...]


def _node_call(partials, node_feat, w1, w2, nrw):
    nb = lambda i: (i, 0)
    wb = lambda i: (0, 0)
    return pl.pallas_call(
        _node_body,
        grid=(5,),
        in_specs=[
            pl.BlockSpec((1, N // 5, D), lambda i: (0, i, 0)),
            pl.BlockSpec((1, N // 5, D), lambda i: (1, i, 0)),
            pl.BlockSpec((N // 5, D), nb),
            pl.BlockSpec((D, D), wb),
            pl.BlockSpec((D, D), wb),
            pl.BlockSpec((1, D), wb),
        ],
        out_specs=pl.BlockSpec((N // 5, D), nb),
        out_shape=jax.ShapeDtypeStruct((N, D), jnp.float32),
    )(partials, partials, node_feat, w1, w2, nrw)


def _edge_fin_body(ssum_ref, cnt_ref, ef_ref, erw_ref, out_ref):
    c = jnp.maximum(cnt_ref[...], 1.0)
    out_ref[...] = ssum_ref[...] / c + erw_ref[...] * ef_ref[...]


def _edge_fin_call(ssum, counts, edge_feat, erw):
    BU = 8000
    ub = lambda i: (i, 0)
    wb = lambda i: (0, 0)
    return pl.pallas_call(
        _edge_fin_body,
        grid=(E_UND // BU,),
        in_specs=[
            pl.BlockSpec((BU, D), ub),
            pl.BlockSpec((BU, 1), ub),
            pl.BlockSpec((BU, D), ub),
            pl.BlockSpec((1, D), wb),
        ],
        out_specs=pl.BlockSpec((BU, D), ub),
        out_shape=jax.ShapeDtypeStruct((E_UND, D), jnp.float32),
    )(ssum, counts, edge_feat, erw)


def kernel(node_feat, edge_feat, smooth_weight, source_index, target_index,
           target_bincount, directed2undirected, Wg, Wv, Wo, ln_scale, ln_bias,
           W1_node, W2_node, W1_edge, W2_edge, W_env, node_res_weight, edge_res_weight):
    wgv = jnp.concatenate([Wg, Wv], axis=1)
    wenv16 = jnp.pad(W_env, ((0, 16 - NB), (0, 0)))
    wenvx = jnp.tile(wenv16, (8, 1))
    sw16 = jnp.pad(smooth_weight, ((0, 0), (0, 16 - NB)))
    swp8 = sw16.reshape(E_UND // 8, D)
    lns = ln_scale.reshape(1, D)
    lnb = ln_bias.reshape(1, D)
    ti1 = target_index.astype(jnp.int32)
    si1 = source_index.astype(jnp.int32)
    du1 = directed2undirected.astype(jnp.int32)
    du8 = du1 // 8
    dul = du1 % 8

    e0, tgt, src, swg = _sc_gather(node_feat, edge_feat, swp8, ti1, si1, du1, du8)

    smooth, delta_edge = _dense_call(e0, tgt, src, swg, dul, wgv, Wo, lns, lnb,
                                     wenvx, W1_edge, W2_edge)

    zeros_nd = jnp.zeros((N_PAD, D), jnp.float32)
    node_partials = _sc_node_scatter(smooth, ti1, zeros_nd)

    seg_sum = jax.ops.segment_sum(delta_edge, directed2undirected, num_segments=E_UND)
    counts = jax.ops.segment_sum(jnp.ones((E_DIR, 1), jnp.float32),
                                 directed2undirected, num_segments=E_UND)

    update_node = _node_call(node_partials, node_feat, W1_node, W2_node, node_res_weight)
    update_edge = _edge_fin_call(seg_sum, counts, edge_feat, edge_res_weight)
    return (update_node, update_edge)


# speedup vs baseline: 2.4153x; 1.0016x over previous
import functools

import jax
import jax.numpy as jnp
from jax import lax
from jax.experimental import pallas as pl
from jax.experimental.pallas import tpu as pltpu
from jax.experimental.pallas import tpu_sc as plsc

N = 10000
E_DIR = 320000
E_UND = 160000
D = 128
H = 128
NB = 7

NC = 2
NS = 16
NW = NC * NS

BE = 2000
N_EBLK = E_DIR // BE


def _silu(x):
    return x * jax.nn.sigmoid(x)


EPW = E_DIR // NW
CI = 1000
CR = 200
N_PAD = 10240
NPT = N_PAD // NS

_g_mesh = plsc.VectorSubcoreMesh(core_axis_name="c", subcore_axis_name="s")


@functools.partial(
    pl.kernel,
    out_type=[
        jax.ShapeDtypeStruct((E_DIR, D), jnp.float32),
        jax.ShapeDtypeStruct((E_DIR, D), jnp.float32),
        jax.ShapeDtypeStruct((E_DIR, D), jnp.float32),
        jax.ShapeDtypeStruct((E_DIR, D), jnp.float32),
    ],
    mesh=_g_mesh,
    scratch_types=[
        pltpu.VMEM((CI,), jnp.int32),
        pltpu.VMEM((CI,), jnp.int32),
        pltpu.VMEM((CI,), jnp.int32),
        pltpu.VMEM((CI,), jnp.int32),
        pltpu.VMEM((CR, D), jnp.float32),
        pltpu.VMEM((CR, D), jnp.float32),
        pltpu.SemaphoreType.DMA,
    ],
)
def _sc_gather(nf_hbm, ef_hbm, sw_hbm, ti_hbm, si_hbm, du_hbm, du8_hbm,
               e0_out, tgt_out, src_out, swg_out,
               ti_v, si_v, du_v, du8_v, rows_v, sw_v, sem):
    cid = lax.axis_index("c")
    sid = lax.axis_index("s")
    wid = sid * NC + cid
    base = wid * EPW

    def outer(j, _):
        off = base + j * CI
        pltpu.sync_copy(ti_hbm.at[pl.ds(off, CI)], ti_v)
        pltpu.sync_copy(si_hbm.at[pl.ds(off, CI)], si_v)
        pltpu.sync_copy(du_hbm.at[pl.ds(off, CI)], du_v)
        pltpu.sync_copy(du8_hbm.at[pl.ds(off, CI)], du8_v)

        def inner(k, _):
            o2 = off + k * CR
            pltpu.async_copy(ef_hbm.at[du_v.at[pl.ds(k * CR, CR)]], rows_v, sem).wait()
            pltpu.sync_copy(rows_v, e0_out.at[pl.ds(o2, CR)])
            pltpu.async_copy(nf_hbm.at[ti_v.at[pl.ds(k * CR, CR)]], rows_v, sem).wait()
            pltpu.sync_copy(rows_v, tgt_out.at[pl.ds(o2, CR)])
            pltpu.async_copy(nf_hbm.at[si_v.at[pl.ds(k * CR, CR)]], rows_v, sem).wait()
            pltpu.sync_copy(rows_v, src_out.at[pl.ds(o2, CR)])
            pltpu.async_copy(sw_hbm.at[du8_v.at[pl.ds(k * CR, CR)]], sw_v, sem).wait()
            pltpu.sync_copy(sw_v, swg_out.at[pl.ds(o2, CR)])
            return 0

        lax.fori_loop(0, CI // CR, inner, 0)
        return 0

    lax.fori_loop(0, EPW // CI, outer, 0)


EPC = E_DIR // NC
EPT = EPC // NS
NST = N_PAD // NS


@functools.partial(
    pl.kernel,
    out_type=jax.ShapeDtypeStruct((NC, N_PAD, D), jnp.float32),
    mesh=_g_mesh,
    scratch_types=[
        pltpu.VMEM((CR,), jnp.int32),
        pltpu.VMEM((CR, D), jnp.float32),
        pltpu.VMEM_SHARED((N_PAD, D), jnp.float32),
        pltpu.SemaphoreType.DMA,
    ],
)
def _sc_node_scatter(smooth_hbm, ti_hbm, zeros_hbm, out_hbm,
                     ti_v, rows_v, acc, sem):
    cid = lax.axis_index("c")
    sid = lax.axis_index("s")
    pltpu.sync_copy(zeros_hbm.at[pl.ds(sid * NST, NST)],
                    acc.at[pl.ds(sid * NST, NST)])
    plsc.subcore_barrier()
    base = cid * EPC + sid * EPT

    def step(j, _):
        off = base + j * CR
        pltpu.sync_copy(ti_hbm.at[pl.ds(off, CR)], ti_v)
        pltpu.sync_copy(smooth_hbm.at[pl.ds(off, CR)], rows_v)
        pltpu.sync_copy(rows_v, acc.at[ti_v], add=True)
        return 0

    lax.fori_loop(0, EPT // CR, step, 0)
    plsc.subcore_barrier()
    pltpu.sync_copy(acc.at[pl.ds(sid * NST, NST)],
                    out_hbm.at[cid].at[pl.ds(sid * NST, NST)])


def _dense_body(e0_ref, tgt_ref, src_ref, swg_ref, dul_ref, wgv_ref, wo_ref,
                lns_ref, lnb_ref, wenv_ref, w1e_ref, w2e_ref,
                smooth_ref, de_ref):
    e0 = e0_ref[...]
    tgt = tgt_ref[...]
    src = src_ref[...]
    wgv = wgv_ref[...]
    gv = (jnp.dot(e0, wgv[0:D], preferred_element_type=jnp.float32)
          + jnp.dot(tgt, wgv[D:2 * D], preferred_element_type=jnp.float32)
          + jnp.dot(src, wgv[2 * D:3 * D], preferred_element_type=jnp.float32))
    g = gv[:, 0:H]
    v = gv[:, H:2 * H]
    h = _silu(g) * v
    ho = jnp.dot(h, wo_ref[...], preferred_element_type=jnp.float32)
    m = jnp.mean(ho, axis=-1, keepdims=True)
    c = ho - m
    var = jnp.mean(c * c, axis=-1, keepdims=True)
    ln = c * jax.lax.rsqrt(var + 1e-5) * lns_ref[...] + lnb_ref[...]
    dul = dul_ref[...].reshape(BE, 1)
    grp = lax.broadcasted_iota(jnp.int32, (BE, D), 1) // 16
    sw_sel = jnp.where(grp == dul, swg_ref[...], 0.0)
    sw = jnp.dot(sw_sel, wenv_ref[...], preferred_element_type=jnp.float32)
    smooth_ref[...] = ln * sw
    de = jnp.dot(_silu(jnp.dot(ln, w1e_ref[...], preferred_element_type=jnp.float32)),
                 w2e_ref[...], preferred_element_type=jnp.float32)
    de_ref[...] = de


def _dense_call(e0, tgt, src, swg, dul, wgv, wo, ln_scale, ln_bias, wenvx, w1e, w2e):
    eb = lambda i: (i, 0)
    ib = lambda i: (i, 0, 0)
    wb = lambda i: (0, 0)
    dul3 = dul.reshape(N_EBLK, 1, BE)
    return pl.pallas_call(
        _dense_body,
        grid=(N_EBLK,),
        in_specs=[
            pl.BlockSpec((BE, D), eb),
            pl.BlockSpec((BE, D), eb),
            pl.BlockSpec((BE, D), eb),
            pl.BlockSpec((BE, D), eb),
            pl.BlockSpec((1, 1, BE), ib),
            pl.BlockSpec((3 * D, 2 * H), wb),
            pl.BlockSpec((H, D), wb),
            pl.BlockSpec((1, D), wb),
            pl.BlockSpec((1, D), wb),
            pl.BlockSpec((D, D), wb),
            pl.BlockSpec((D, D), wb),
            pl.BlockSpec((D, D), wb),
        ],
        out_specs=[pl.BlockSpec((BE, D), eb), pl.BlockSpec((BE, D), eb)],
        out_shape=[
            jax.ShapeDtypeStruct((E_DIR, D), jnp.float32),
            jax.ShapeDtypeStruct((E_DIR, D), jnp.float32),
        ],
    )(e0, tgt, src, swg, dul3, wgv, wo, ln_scale, ln_bias, wenvx, w1e, w2e)


def _node_body(r0_ref, r1_ref, nf_ref, w1_ref, w2_ref, nrw_ref, out_ref):
    r = r0_ref[0] + r1_ref[0]
    d = jnp.dot(_silu(jnp.dot(r, w1_ref[...], preferred_element_type=jnp.float32)),
                w2_ref[...], preferred_element_type=jnp.float32)
    out_ref[...] = d + nrw_ref[...] * nf_ref[...]


def _node_call(partials, node_feat, w1, w2, nrw):
    nb = lambda i: (i, 0)
    wb = lambda i: (0, 0)
    return pl.pallas_call(
        _node_body,
        grid=(5,),
        in_specs=[
            pl.BlockSpec((1, N // 5, D), lambda i: (0, i, 0)),
            pl.BlockSpec((1, N // 5, D), lambda i: (1, i, 0)),
            pl.BlockSpec((N // 5, D), nb),
            pl.BlockSpec((D, D), wb),
            pl.BlockSpec((D, D), wb),
            pl.BlockSpec((1, D), wb),
        ],
        out_specs=pl.BlockSpec((N // 5, D), nb),
        out_shape=jax.ShapeDtypeStruct((N, D), jnp.float32),
    )(partials, partials, node_feat, w1, w2, nrw)


BU = 8000


def _edge_fin_body(ssum_ref, cnt_ref, ef_ref, erw_ref, out_ref):
    c = jnp.maximum(cnt_ref[...], 1.0)
    out_ref[...] = ssum_ref[...] / c + erw_ref[...] * ef_ref[...]


def _edge_fin_call(ssum, cnts, edge_feat, erw):
    ub = lambda i: (i, 0)
    wb = lambda i: (0, 0)
    return pl.pallas_call(
        _edge_fin_body,
        grid=(E_UND // BU,),
        in_specs=[
            pl.BlockSpec((BU, D), ub),
            pl.BlockSpec((BU, 1), ub),
            pl.BlockSpec((BU, D), ub),
            pl.BlockSpec((1, D), wb),
        ],
        out_specs=pl.BlockSpec((BU, D), ub),
        out_shape=jax.ShapeDtypeStruct((E_UND, D), jnp.float32),
    )(ssum, cnts, edge_feat, erw)


def kernel(node_feat, edge_feat, smooth_weight, source_index, target_index,
           target_bincount, directed2undirected, Wg, Wv, Wo, ln_scale, ln_bias,
           W1_node, W2_node, W1_edge, W2_edge, W_env, node_res_weight, edge_res_weight):
    wgv = jnp.concatenate([Wg, Wv], axis=1)
    wenv16 = jnp.pad(W_env, ((0, 16 - NB), (0, 0)))
    wenvx = jnp.tile(wenv16, (8, 1))
    sw16 = jnp.pad(smooth_weight, ((0, 0), (0, 16 - NB)))
    swp8 = sw16.reshape(E_UND // 8, D)
    lns = ln_scale.reshape(1, D)
    lnb = ln_bias.reshape(1, D)
    ti1 = target_index.astype(jnp.int32)
    si1 = source_index.astype(jnp.int32)
    du1 = directed2undirected.astype(jnp.int32)
    du8 = du1 // 8
    dul = du1 % 8

    e0, tgt, src, swg = _sc_gather(node_feat, edge_feat, swp8, ti1, si1, du1, du8)

    smooth, delta_edge = _dense_call(e0, tgt, src, swg, dul, wgv, Wo, lns, lnb,
                                     wenvx, W1_edge, W2_edge)

    zeros_nd = jnp.zeros((N_PAD, D), jnp.float32)
    node_partials = _sc_node_scatter(smooth, ti1, zeros_nd)

    ssum = jax.ops.segment_sum(delta_edge, du1, num_segments=E_UND)
    cnts = jax.ops.segment_sum(jnp.ones((E_DIR, 1), jnp.float32), du1,
                               num_segments=E_UND)

    update_node = _node_call(node_partials, node_feat, W1_node, W2_node, node_res_weight)
    update_edge = _edge_fin_call(ssum, cnts, edge_feat, edge_res_weight)
    return (update_node, update_edge)


# speedup vs baseline: 2.5721x; 1.0649x over previous
import functools

import jax
import jax.numpy as jnp
from jax import lax
from jax.experimental import pallas as pl
from jax.experimental.pallas import tpu as pltpu
from jax.experimental.pallas import tpu_sc as plsc

N = 10000
E_DIR = 320000
E_UND = 160000
D = 128
H = 128
NB = 7

NC = 2
NS = 16
NW = NC * NS

BE = 2000
N_EBLK = E_DIR // BE


def _silu(x):
    return x * jax.nn.sigmoid(x)


EPW = E_DIR // NW
CI = 2000
CR = 400
N_PAD = 10240
NPT = N_PAD // NS

_g_mesh = plsc.VectorSubcoreMesh(core_axis_name="c", subcore_axis_name="s")


@functools.partial(
    pl.kernel,
    out_type=[
        jax.ShapeDtypeStruct((E_DIR, D), jnp.float32),
        jax.ShapeDtypeStruct((E_DIR, D), jnp.float32),
        jax.ShapeDtypeStruct((E_DIR, D), jnp.float32),
        jax.ShapeDtypeStruct((E_DIR, D), jnp.float32),
    ],
    mesh=_g_mesh,
    scratch_types=[
        pltpu.VMEM((CI,), jnp.int32),
        pltpu.VMEM((CI,), jnp.int32),
        pltpu.VMEM((CI,), jnp.int32),
        pltpu.VMEM((CI,), jnp.int32),
        pltpu.VMEM((CR, D), jnp.float32),
        pltpu.VMEM((CR, D), jnp.float32),
        pltpu.SemaphoreType.DMA,
    ],
)
def _sc_gather(nf_hbm, ef_hbm, sw_hbm, ti_hbm, si_hbm, du_hbm, du8_hbm,
               e0_out, tgt_out, src_out, swg_out,
               ti_v, si_v, du_v, du8_v, rows_v, sw_v, sem):
    cid = lax.axis_index("c")
    sid = lax.axis_index("s")
    wid = sid * NC + cid
    base = wid * EPW

    def outer(j, _):
        off = base + j * CI
        pltpu.sync_copy(ti_hbm.at[pl.ds(off, CI)], ti_v)
        pltpu.sync_copy(si_hbm.at[pl.ds(off, CI)], si_v)
        pltpu.sync_copy(du_hbm.at[pl.ds(off, CI)], du_v)
        pltpu.sync_copy(du8_hbm.at[pl.ds(off, CI)], du8_v)

        def inner(k, _):
            o2 = off + k * CR
            pltpu.async_copy(ef_hbm.at[du_v.at[pl.ds(k * CR, CR)]], rows_v, sem).wait()
            pltpu.sync_copy(rows_v, e0_out.at[pl.ds(o2, CR)])
            pltpu.async_copy(nf_hbm.at[ti_v.at[pl.ds(k * CR, CR)]], rows_v, sem).wait()
            pltpu.sync_copy(rows_v, tgt_out.at[pl.ds(o2, CR)])
            pltpu.async_copy(nf_hbm.at[si_v.at[pl.ds(k * CR, CR)]], rows_v, sem).wait()
            pltpu.sync_copy(rows_v, src_out.at[pl.ds(o2, CR)])
            pltpu.async_copy(sw_hbm.at[du8_v.at[pl.ds(k * CR, CR)]], sw_v, sem).wait()
            pltpu.sync_copy(sw_v, swg_out.at[pl.ds(o2, CR)])
            return 0

        lax.fori_loop(0, CI // CR, inner, 0)
        return 0

    lax.fori_loop(0, EPW // CI, outer, 0)


EPC = E_DIR // NC
EPT = EPC // NS
NST = N_PAD // NS
CRN = 200


@functools.partial(
    pl.kernel,
    out_type=jax.ShapeDtypeStruct((NC, N_PAD, D), jnp.float32),
    mesh=_g_mesh,
    scratch_types=[
        pltpu.VMEM((CRN,), jnp.int32),
        pltpu.VMEM((CRN, D), jnp.float32),
        pltpu.VMEM_SHARED((N_PAD, D), jnp.float32),
        pltpu.SemaphoreType.DMA,
    ],
)
def _sc_node_scatter(smooth_hbm, ti_hbm, zeros_hbm, out_hbm,
                     ti_v, rows_v, acc, sem):
    cid = lax.axis_index("c")
    sid = lax.axis_index("s")
    pltpu.sync_copy(zeros_hbm.at[pl.ds(sid * NST, NST)],
                    acc.at[pl.ds(sid * NST, NST)])
    plsc.subcore_barrier()
    base = cid * EPC + sid * EPT

    def step(j, _):
        off = base + j * CRN
        pltpu.sync_copy(ti_hbm.at[pl.ds(off, CRN)], ti_v)
        pltpu.sync_copy(smooth_hbm.at[pl.ds(off, CRN)], rows_v)
        pltpu.sync_copy(rows_v, acc.at[ti_v], add=True)
        return 0

    lax.fori_loop(0, EPT // CRN, step, 0)
    plsc.subcore_barrier()
    pltpu.sync_copy(acc.at[pl.ds(sid * NST, NST)],
                    out_hbm.at[cid].at[pl.ds(sid * NST, NST)])


def _dense_body(e0_ref, tgt_ref, src_ref, swg_ref, dul_ref, wgv_ref, wo_ref,
                lns_ref, lnb_ref, wenv_ref, w1e_ref, w2e_ref,
                smooth_ref, de_ref):
    e0 = e0_ref[...]
    tgt = tgt_ref[...]
    src = src_ref[...]
    wgv = wgv_ref[...]
    gv = (jnp.dot(e0, wgv[0:D], preferred_element_type=jnp.float32)
          + jnp.dot(tgt, wgv[D:2 * D], preferred_element_type=jnp.float32)
          + jnp.dot(src, wgv[2 * D:3 * D], preferred_element_type=jnp.float32))
    g = gv[:, 0:H]
    v = gv[:, H:2 * H]
    h = _silu(g) * v
    ho = jnp.dot(h, wo_ref[...], preferred_element_type=jnp.float32)
    m = jnp.mean(ho, axis=-1, keepdims=True)
    c = ho - m
    var = jnp.mean(c * c, axis=-1, keepdims=True)
    ln = c * jax.lax.rsqrt(var + 1e-5) * lns_ref[...] + lnb_ref[...]
    dul = dul_ref[...].reshape(BE, 1)
    grp = lax.broadcasted_iota(jnp.int32, (BE, D), 1) // 16
    sw_sel = jnp.where(grp == dul, swg_ref[...], 0.0)
    sw = jnp.dot(sw_sel, wenv_ref[...], preferred_element_type=jnp.float32)
    smooth_ref[...] = ln * sw
    de = jnp.dot(_silu(jnp.dot(ln, w1e_ref[...], preferred_element_type=jnp.float32)),
                 w2e_ref[...], preferred_element_type=jnp.float32)
    de_ref[...] = de


def _dense_call(e0, tgt, src, swg, dul, wgv, wo, ln_scale, ln_bias, wenvx, w1e, w2e):
    eb = lambda i: (i, 0)
    ib = lambda i: (i, 0, 0)
    wb = lambda i: (0, 0)
    dul3 = dul.reshape(N_EBLK, 1, BE)
    return pl.pallas_call(
        _dense_body,
        grid=(N_EBLK,),
        in_specs=[
            pl.BlockSpec((BE, D), eb),
            pl.BlockSpec((BE, D), eb),
            pl.BlockSpec((BE, D), eb),
            pl.BlockSpec((BE, D), eb),
            pl.BlockSpec((1, 1, BE), ib),
            pl.BlockSpec((3 * D, 2 * H), wb),
            pl.BlockSpec((H, D), wb),
            pl.BlockSpec((1, D), wb),
            pl.BlockSpec((1, D), wb),
            pl.BlockSpec((D, D), wb),
            pl.BlockSpec((D, D), wb),
            pl.BlockSpec((D, D), wb),
        ],
        out_specs=[pl.BlockSpec((BE, D), eb), pl.BlockSpec((BE, D), eb)],
        out_shape=[
            jax.ShapeDtypeStruct((E_DIR, D), jnp.float32),
            jax.ShapeDtypeStruct((E_DIR, D), jnp.float32),
        ],
    )(e0, tgt, src, swg, dul3, wgv, wo, ln_scale, ln_bias, wenvx, w1e, w2e)


def _node_body(r0_ref, r1_ref, nf_ref, w1_ref, w2_ref, nrw_ref, out_ref):
    r = r0_ref[0] + r1_ref[0]
    d = jnp.dot(_silu(jnp.dot(r, w1_ref[...], preferred_element_type=jnp.float32)),
                w2_ref[...], preferred_element_type=jnp.float32)
    out_ref[...] = d + nrw_ref[...] * nf_ref[...]


def _node_call(partials, node_feat, w1, w2, nrw):
    nb = lambda i: (i, 0)
    wb = lambda i: (0, 0)
    return pl.pallas_call(
        _node_body,
        grid=(5,),
        in_specs=[
            pl.BlockSpec((1, N // 5, D), lambda i: (0, i, 0)),
            pl.BlockSpec((1, N // 5, D), lambda i: (1, i, 0)),
            pl.BlockSpec((N // 5, D), nb),
            pl.BlockSpec((D, D), wb),
            pl.BlockSpec((D, D), wb),
            pl.BlockSpec((1, D), wb),
        ],
        out_specs=pl.BlockSpec((N // 5, D), nb),
        out_shape=jax.ShapeDtypeStruct((N, D), jnp.float32),
    )(partials, partials, node_feat, w1, w2, nrw)


BU = 8000


def _edge_fin_body(ssum_ref, cnt_ref, ef_ref, erw_ref, out_ref):
    c = jnp.maximum(cnt_ref[...], 1.0)
    out_ref[...] = ssum_ref[...] / c + erw_ref[...] * ef_ref[...]


def _edge_fin_call(ssum, cnts, edge_feat, erw):
    ub = lambda i: (i, 0)
    wb = lambda i: (0, 0)
    return pl.pallas_call(
        _edge_fin_body,
        grid=(E_UND // BU,),
        in_specs=[
            pl.BlockSpec((BU, D), ub),
            pl.BlockSpec((BU, 1), ub),
            pl.BlockSpec((BU, D), ub),
            pl.BlockSpec((1, D), wb),
        ],
        out_specs=pl.BlockSpec((BU, D), ub),
        out_shape=jax.ShapeDtypeStruct((E_UND, D), jnp.float32),
    )(ssum, cnts, edge_feat, erw)


def kernel(node_feat, edge_feat, smooth_weight, source_index, target_index,
           target_bincount, directed2undirected, Wg, Wv, Wo, ln_scale, ln_bias,
           W1_node, W2_node, W1_edge, W2_edge, W_env, node_res_weight, edge_res_weight):
    wgv = jnp.concatenate([Wg, Wv], axis=1)
    wenv16 = jnp.pad(W_env, ((0, 16 - NB), (0, 0)))
    wenvx = jnp.tile(wenv16, (8, 1))
    sw16 = jnp.pad(smooth_weight, ((0, 0), (0, 16 - NB)))
    swp8 = sw16.reshape(E_UND // 8, D)
    lns = ln_scale.reshape(1, D)
    lnb = ln_bias.reshape(1, D)
    ti1 = target_index.astype(jnp.int32)
    si1 = source_index.astype(jnp.int32)
    du1 = directed2undirected.astype(jnp.int32)
    du8 = du1 // 8
    dul = du1 % 8

    e0, tgt, src, swg = _sc_gather(node_feat, edge_feat, swp8, ti1, si1, du1, du8)

    smooth, delta_edge = _dense_call(e0, tgt, src, swg, dul, wgv, Wo, lns, lnb,
                                     wenvx, W1_edge, W2_edge)

    zeros_nd = jnp.zeros((N_PAD, D), jnp.float32)
    node_partials = _sc_node_scatter(smooth, ti1, zeros_nd)

    ssum = jax.ops.segment_sum(delta_edge, du1, num_segments=E_UND)
    cnts = jax.ops.segment_sum(jnp.ones((E_DIR, 1), jnp.float32), du1,
                               num_segments=E_UND)

    update_node = _node_call(node_partials, node_feat, W1_node, W2_node, node_res_weight)
    update_edge = _edge_fin_call(ssum, cnts, edge_feat, edge_res_weight)
    return (update_node, update_edge)
